# baseline (speedup 1.0000x reference)
import functools

import jax
import jax.numpy as jnp
from jax import lax
from jax.experimental import pallas as pl
from jax.experimental.pallas import tpu as pltpu
from jax.experimental.pallas import tpu_sc as plsc

N_GRAPH, N_NODE, N_FEAT = 1024, 128, 9
HIDDEN = 64
NUM_WORKERS = 32
GRAPHS_PER_WORKER = N_GRAPH // NUM_WORKERS
LANES = 16
VPR = HIDDEN // LANES
HALF = N_NODE // 2
N_ADD = 4
N_VEC = N_FEAT - N_ADD
ROWS_PER_HALF = HALF * N_VEC


def _sc_kernel():
    mesh = plsc.VectorSubcoreMesh(core_axis_name="c", subcore_axis_name="s")

    @functools.partial(
        pl.kernel,
        mesh=mesh,
        out_type=jax.ShapeDtypeStruct((N_GRAPH, N_NODE + 1, HIDDEN), jnp.float32),
        scratch_types=[
            pltpu.VMEM((2, N_FEAT, HALF), jnp.int32),
            pltpu.VMEM((2, 2, HALF), jnp.int32),
            pltpu.VMEM((2, ROWS_PER_HALF, HIDDEN), jnp.float32),
            pltpu.VMEM((2, 2, HALF, HIDDEN), jnp.float32),
            pltpu.VMEM((2, HALF, HIDDEN), jnp.float32),
            pltpu.VMEM((N_NODE + 1, HIDDEN), jnp.float32),
            pltpu.SemaphoreType.DMA,
            pltpu.SemaphoreType.DMA,
            pltpu.SemaphoreType.DMA,
            pltpu.SemaphoreType.DMA,
            pltpu.SemaphoreType.DMA,
            pltpu.SemaphoreType.DMA,
            pltpu.SemaphoreType.DMA,
        ],
        compiler_params=pltpu.CompilerParams(use_tc_tiling_on_sc=False),
    )
    def k(x_p, in_deg, out_deg, atom_t, in_t, out_t, token, zeros,
          out, idxa, idxd, rows, drows, acc, obuf,
          sem0, sem1, semw, semi0, semi1, semz0, semz1):
        wid = lax.axis_index("s") * 2 + lax.axis_index("c")
        sems = (sem0, sem1)
        semis = (semi0, semi1)
        semzs = (semz0, semz1)

        def prep(g, s, h):
            semi, semz = semis[s], semzs[s]
            pltpu.async_copy(zeros, acc.at[s], semz)
            for j in range(N_FEAT):
                pltpu.async_copy(x_p.at[j, g, pl.ds(h * HALF, HALF)],
                                 idxa.at[s, j], semi)
            pltpu.async_copy(in_deg.at[g, pl.ds(h * HALF, HALF)],
                             idxd.at[s, 0], semi)
            pltpu.async_copy(out_deg.at[g, pl.ds(h * HALF, HALF)],
                             idxd.at[s, 1], semi)
            pltpu.make_async_copy(zeros, acc.at[s], semz).wait()
            for j in range(N_FEAT):
                pltpu.make_async_copy(x_p.at[j, g, pl.ds(h * HALF, HALF)],
                                      idxa.at[s, j], semi).wait()
            pltpu.make_async_copy(in_deg.at[g, pl.ds(h * HALF, HALF)],
                                  idxd.at[s, 0], semi).wait()
            pltpu.make_async_copy(out_deg.at[g, pl.ds(h * HALF, HALF)],
                                  idxd.at[s, 1], semi).wait()

        def fire(s):
            sem = sems[s]
            for j in range(N_ADD):
                pltpu.async_copy(atom_t.at[idxa.at[s, j]], acc.at[s],
                                 sem, add=True)
            for j in range(N_VEC):
                pltpu.async_copy(atom_t.at[idxa.at[s, N_ADD + j]],
                                 rows.at[s, pl.ds(j * HALF, HALF)], sem)
            pltpu.async_copy(in_t.at[idxd.at[s, 0]], drows.at[s, 0], sem)
            pltpu.async_copy(out_t.at[idxd.at[s, 1]], drows.at[s, 1], sem)

        def drain(s):
            sem = sems[s]
            for j in range(N_ADD):
                pltpu.make_async_copy(atom_t.at[idxa.at[s, j]], acc.at[s],
                                      sem).wait()
            for j in range(N_VEC):
                pltpu.make_async_copy(atom_t.at[idxa.at[s, N_ADD + j]],
                                      rows.at[s, pl.ds(j * HALF, HALF)],
                                      sem).wait()
            pltpu.make_async_copy(in_t.at[idxd.at[s, 0]], drows.at[s, 0], sem).wait()
            pltpu.make_async_copy(out_t.at[idxd.at[s, 1]], drows.at[s, 1], sem).wait()

        def sum_half(s, h):
            base = h * HALF + 1

            def per_node(n, nc):
                cols = []
                for v in range(VPR):
                    sl = pl.ds(v * LANES, LANES)
                    vals = [acc[s, n, sl]]
                    vals += [rows[s, j * HALF + n, sl] for j in range(N_VEC)]
                    vals.append(drows[s, 0, n, sl])
                    vals.append(drows[s, 1, n, sl])
                    cols.append(vals)
                outs = []
                for vals in cols:
                    while len(vals) > 1:
                        nxt = [vals[i] + vals[i + 1]
                               for i in range(0, len(vals) - 1, 2)]
                        if len(vals) % 2:
                            nxt.append(vals[-1])
                        vals = nxt
                    outs.append(vals[0])
                for v in range(VPR):
                    obuf[base + n, pl.ds(v * LANES, LANES)] = outs[v]
                return nc

            lax.fori_loop(0, HALF, per_node, 0)

        pltpu.sync_copy(token, obuf.at[pl.ds(0, 1)])
        g0 = wid * GRAPHS_PER_WORKER
        prep(g0, 0, 0)
        fire(0)

        def per_graph(t, carry):
            g = g0 + t
            prep(g, 1, 1)
            fire(1)
            drain(0)

            @pl.when(t > 0)
            def _():
                pltpu.make_async_copy(obuf, out.at[g - 1], semw).wait()

            sum_half(0, 0)

            @pl.when(t < GRAPHS_PER_WORKER - 1)
            def _():
                prep(g + 1, 0, 0)
                fire(0)

            drain(1)
            sum_half(1, 1)
            pltpu.async_copy(obuf, out.at[g], semw)
            return carry

        lax.fori_loop(0, GRAPHS_PER_WORKER, per_graph, 0)
        pltpu.make_async_copy(
            obuf, out.at[g0 + GRAPHS_PER_WORKER - 1], semw).wait()

    return k


def kernel(x, in_degree, out_degree, atom_table, in_table, out_table, graph_token_w):
    x_p = jnp.transpose(x.astype(jnp.int32), (2, 0, 1))
    zeros = jnp.zeros((HALF, HIDDEN), jnp.float32)
    return _sc_kernel()(
        x_p,
        in_degree.astype(jnp.int32),
        out_degree.astype(jnp.int32),
        atom_table,
        in_table,
        out_table,
        graph_token_w,
        zeros,
    )


# speedup vs baseline: 1.0476x; 1.0476x over previous
import functools

import jax
import jax.numpy as jnp
from jax import lax
from jax.experimental import pallas as pl
from jax.experimental.pallas import tpu as pltpu
from jax.experimental.pallas import tpu_sc as plsc

N_GRAPH, N_NODE, N_FEAT = 1024, 128, 9
HIDDEN = 64
NUM_WORKERS = 32
GRAPHS_PER_WORKER = N_GRAPH // NUM_WORKERS
LANES = 16
VPR = HIDDEN // LANES
HALF = N_NODE // 2
ROWS_PER_HALF = HALF * N_FEAT


def _sc_kernel():
    mesh = plsc.VectorSubcoreMesh(core_axis_name="c", subcore_axis_name="s")

    @functools.partial(
        pl.kernel,
        mesh=mesh,
        out_type=jax.ShapeDtypeStruct((N_GRAPH, N_NODE + 1, HIDDEN), jnp.float32),
        scratch_types=[
            pltpu.VMEM((2, N_FEAT, HALF), jnp.int32),
            pltpu.VMEM((2, 2, HALF), jnp.int32),
            pltpu.VMEM((2, ROWS_PER_HALF, HIDDEN), jnp.bfloat16),
            pltpu.VMEM((2, 2, HALF, HIDDEN), jnp.float32),
            pltpu.VMEM((N_NODE + 1, HIDDEN), jnp.float32),
            pltpu.SemaphoreType.DMA,
            pltpu.SemaphoreType.DMA,
            pltpu.SemaphoreType.DMA,
            pltpu.SemaphoreType.DMA,
            pltpu.SemaphoreType.DMA,
        ],
        compiler_params=pltpu.CompilerParams(
            use_tc_tiling_on_sc=False, needs_layout_passes=False),
    )
    def k(x_p, in_deg, out_deg, atom_t, in_t, out_t, token,
          out, idxa, idxd, rows, drows, obuf, sem0, sem1, semw, semi0, semi1):
        wid = lax.axis_index("s") * 2 + lax.axis_index("c")
        sems = (sem0, sem1)
        semis = (semi0, semi1)

        def load_idx(g, s, h):
            semi = semis[s]
            for j in range(N_FEAT):
                pltpu.async_copy(x_p.at[j, g, pl.ds(h * HALF, HALF)],
                                 idxa.at[s, j], semi)
            pltpu.async_copy(in_deg.at[g, pl.ds(h * HALF, HALF)],
                             idxd.at[s, 0], semi)
            pltpu.async_copy(out_deg.at[g, pl.ds(h * HALF, HALF)],
                             idxd.at[s, 1], semi)
            for j in range(N_FEAT):
                pltpu.make_async_copy(x_p.at[j, g, pl.ds(h * HALF, HALF)],
                                      idxa.at[s, j], semi).wait()
            pltpu.make_async_copy(in_deg.at[g, pl.ds(h * HALF, HALF)],
                                  idxd.at[s, 0], semi).wait()
            pltpu.make_async_copy(out_deg.at[g, pl.ds(h * HALF, HALF)],
                                  idxd.at[s, 1], semi).wait()

        def fire(s):
            sem = sems[s]
            for j in range(N_FEAT):
                pltpu.async_copy(atom_t.at[idxa.at[s, j]],
                                 rows.at[s, pl.ds(j * HALF, HALF)], sem)
            pltpu.async_copy(in_t.at[idxd.at[s, 0]], drows.at[s, 0], sem)
            pltpu.async_copy(out_t.at[idxd.at[s, 1]], drows.at[s, 1], sem)

        def drain(s):
            sem = sems[s]
            for j in range(N_FEAT):
                pltpu.make_async_copy(atom_t.at[idxa.at[s, j]],
                                      rows.at[s, pl.ds(j * HALF, HALF)], sem).wait()
            pltpu.make_async_copy(in_t.at[idxd.at[s, 0]], drows.at[s, 0], sem).wait()
            pltpu.make_async_copy(out_t.at[idxd.at[s, 1]], drows.at[s, 1], sem).wait()

        def sum_half(s, h):
            base = h * HALF + 1

            mask = jnp.full((LANES,), -65536, jnp.int32)

            def per_node(n, nc):
                cols = [[] for _ in range(VPR)]
                for blk in range(VPR // 2):
                    for j in range(N_FEAT):
                        v32 = rows[s, j * HALF + n, pl.ds(blk * 2 * LANES, 2 * LANES)]
                        vi = plsc.bitcast(v32, jnp.int32)
                        lo = plsc.bitcast(jnp.left_shift(vi, 16), jnp.float32)
                        hi = plsc.bitcast(jnp.bitwise_and(vi, mask), jnp.float32)
                        cols[blk * 2].append(lo)
                        cols[blk * 2 + 1].append(hi)
                for v in range(VPR):
                    sl = pl.ds(v * LANES, LANES)
                    cols[v].append(drows[s, 0, n, sl])
                    cols[v].append(drows[s, 1, n, sl])
                outs = []
                for vals in cols:
                    while len(vals) > 1:
                        nxt = [vals[i] + vals[i + 1]
                               for i in range(0, len(vals) - 1, 2)]
                        if len(vals) % 2:
                            nxt.append(vals[-1])
                        vals = nxt
                    outs.append(vals[0])
                for v in range(VPR):
                    obuf[base + n, pl.ds(v * LANES, LANES)] = outs[v]
                return nc

            lax.fori_loop(0, HALF, per_node, 0)

        pltpu.sync_copy(token, obuf.at[pl.ds(0, 1)])
        g0 = wid * GRAPHS_PER_WORKER
        load_idx(g0, 0, 0)
        fire(0)

        def per_graph(t, carry):
            g = g0 + t
            load_idx(g, 1, 1)
            fire(1)
            drain(0)

            @pl.when(t > 0)
            def _():
                pltpu.make_async_copy(obuf, out.at[g - 1], semw).wait()

            sum_half(0, 0)

            @pl.when(t < GRAPHS_PER_WORKER - 1)
            def _():
                load_idx(g + 1, 0, 0)
                fire(0)

            drain(1)
            sum_half(1, 1)
            pltpu.async_copy(obuf, out.at[g], semw)
            return carry

        lax.fori_loop(0, GRAPHS_PER_WORKER, per_graph, 0)
        pltpu.make_async_copy(
            obuf, out.at[g0 + GRAPHS_PER_WORKER - 1], semw).wait()

    return k


def kernel(x, in_degree, out_degree, atom_table, in_table, out_table, graph_token_w):
    x_p = jnp.transpose(x.astype(jnp.int32), (2, 0, 1))
    atom_bf = atom_table.astype(jnp.bfloat16)
    atom_bf = atom_bf.reshape(-1, VPR // 2, 2, LANES)
    atom_bf = jnp.transpose(atom_bf, (0, 1, 3, 2)).reshape(-1, HIDDEN)
    return _sc_kernel()(
        x_p,
        in_degree.astype(jnp.int32),
        out_degree.astype(jnp.int32),
        atom_bf,
        in_table,
        out_table,
        graph_token_w,
    )


# speedup vs baseline: 1.3282x; 1.2677x over previous
import functools

import jax
import jax.numpy as jnp
from jax import lax
from jax.experimental import pallas as pl
from jax.experimental.pallas import tpu as pltpu
from jax.experimental.pallas import tpu_sc as plsc

N_GRAPH, N_NODE, N_FEAT = 1024, 128, 9
HIDDEN = 64
NUM_WORKERS = 32
GRAPHS_PER_WORKER = N_GRAPH // NUM_WORKERS
LANES = 16
VPR = HIDDEN // LANES
HALF = N_NODE // 2
ROWS_PER_HALF = HALF * N_FEAT


def _sc_kernel():
    mesh = plsc.VectorSubcoreMesh(core_axis_name="c", subcore_axis_name="s")

    @functools.partial(
        pl.kernel,
        mesh=mesh,
        out_type=jax.ShapeDtypeStruct((N_GRAPH, N_NODE + 1, HIDDEN), jnp.float32),
        scratch_types=[
            pltpu.VMEM((2, N_FEAT, HALF), jnp.int32),
            pltpu.VMEM((2, 2, HALF), jnp.int32),
            pltpu.VMEM((2, ROWS_PER_HALF, HIDDEN), jnp.float32),
            pltpu.VMEM((2, 2, HALF, HIDDEN), jnp.float32),
            pltpu.VMEM((N_NODE + 1, HIDDEN), jnp.float32),
            pltpu.SemaphoreType.DMA,
            pltpu.SemaphoreType.DMA,
            pltpu.SemaphoreType.DMA,
            pltpu.SemaphoreType.DMA,
            pltpu.SemaphoreType.DMA,
        ],
        compiler_params=pltpu.CompilerParams(use_tc_tiling_on_sc=False),
    )
    def k(x_p, in_deg, out_deg, atom_t, in_t, out_t, token,
          out, idxa, idxd, rows, drows, obuf, sem0, sem1, semw, semi0, semi1):
        wid = lax.axis_index("s") * 2 + lax.axis_index("c")
        sems = (sem0, sem1)
        semis = (semi0, semi1)

        def load_idx(g, s, h):
            semi = semis[s]
            for j in range(N_FEAT):
                pltpu.async_copy(x_p.at[j, g, pl.ds(h * HALF, HALF)],
                                 idxa.at[s, j], semi)
            pltpu.async_copy(in_deg.at[g, pl.ds(h * HALF, HALF)],
                             idxd.at[s, 0], semi)
            pltpu.async_copy(out_deg.at[g, pl.ds(h * HALF, HALF)],
                             idxd.at[s, 1], semi)
            for j in range(N_FEAT):
                pltpu.make_async_copy(x_p.at[j, g, pl.ds(h * HALF, HALF)],
                                      idxa.at[s, j], semi).wait()
            pltpu.make_async_copy(in_deg.at[g, pl.ds(h * HALF, HALF)],
                                  idxd.at[s, 0], semi).wait()
            pltpu.make_async_copy(out_deg.at[g, pl.ds(h * HALF, HALF)],
                                  idxd.at[s, 1], semi).wait()

        def fire(s):
            sem = sems[s]
            for j in range(N_FEAT):
                pltpu.async_copy(atom_t.at[idxa.at[s, j]],
                                 rows.at[s, pl.ds(j * HALF, HALF)], sem)
            pltpu.async_copy(in_t.at[idxd.at[s, 0]], drows.at[s, 0], sem)
            pltpu.async_copy(out_t.at[idxd.at[s, 1]], drows.at[s, 1], sem)

        def drain(s):
            sem = sems[s]
            for j in range(N_FEAT):
                pltpu.make_async_copy(atom_t.at[idxa.at[s, j]],
                                      rows.at[s, pl.ds(j * HALF, HALF)], sem).wait()
            pltpu.make_async_copy(in_t.at[idxd.at[s, 0]], drows.at[s, 0], sem).wait()
            pltpu.make_async_copy(out_t.at[idxd.at[s, 1]], drows.at[s, 1], sem).wait()

        def sum_half(s, h):
            base = h * HALF + 1

            def per_node(n, nc):
                cols = []
                for v in range(VPR):
                    sl = pl.ds(v * LANES, LANES)
                    vals = [rows[s, j * HALF + n, sl] for j in range(N_FEAT)]
                    vals.append(drows[s, 0, n, sl])
                    vals.append(drows[s, 1, n, sl])
                    cols.append(vals)
                outs = []
                for vals in cols:
                    while len(vals) > 1:
                        nxt = [vals[i] + vals[i + 1]
                               for i in range(0, len(vals) - 1, 2)]
                        if len(vals) % 2:
                            nxt.append(vals[-1])
                        vals = nxt
                    outs.append(vals[0])
                for v in range(VPR):
                    obuf[base + n, pl.ds(v * LANES, LANES)] = outs[v]
                return nc

            lax.fori_loop(0, HALF, per_node, 0)

        pltpu.sync_copy(token, obuf.at[pl.ds(0, 1)])
        g0 = wid * GRAPHS_PER_WORKER
        load_idx(g0, 0, 0)
        fire(0)

        def per_graph(t, carry):
            g = g0 + t
            load_idx(g, 1, 1)
            fire(1)
            drain(0)

            @pl.when(t > 0)
            def _():
                pltpu.make_async_copy(obuf, out.at[g - 1], semw).wait()

            sum_half(0, 0)

            @pl.when(t < GRAPHS_PER_WORKER - 1)
            def _():
                load_idx(g + 1, 0, 0)
                fire(0)

            drain(1)
            sum_half(1, 1)
            pltpu.async_copy(obuf, out.at[g], semw)
            return carry

        lax.fori_loop(0, GRAPHS_PER_WORKER, per_graph, 0)
        pltpu.make_async_copy(
            obuf, out.at[g0 + GRAPHS_PER_WORKER - 1], semw).wait()

    return k


def kernel(x, in_degree, out_degree, atom_table, in_table, out_table, graph_token_w):
    x_p = jnp.transpose(x.astype(jnp.int32), (2, 0, 1))
    return _sc_kernel()(
        x_p,
        in_degree.astype(jnp.int32),
        out_degree.astype(jnp.int32),
        atom_table,
        in_table,
        out_table,
        graph_token_w,
    )


# speedup vs baseline: 1.3314x; 1.0025x over previous
import functools

import jax
import jax.numpy as jnp
from jax import lax
from jax.experimental import pallas as pl
from jax.experimental.pallas import tpu as pltpu
from jax.experimental.pallas import tpu_sc as plsc

N_GRAPH, N_NODE, N_FEAT = 1024, 128, 9
HIDDEN = 64
NUM_WORKERS = 32
GRAPHS_PER_WORKER = N_GRAPH // NUM_WORKERS
LANES = 16
VPR = HIDDEN // LANES
HALF = N_NODE // 2
ROWS_PER_HALF = HALF * N_FEAT


def _sc_kernel():
    mesh = plsc.VectorSubcoreMesh(core_axis_name="c", subcore_axis_name="s")

    @functools.partial(
        pl.kernel,
        mesh=mesh,
        out_type=jax.ShapeDtypeStruct((N_GRAPH, N_NODE + 1, HIDDEN), jnp.float32),
        scratch_types=[
            pltpu.VMEM((2, N_FEAT, HALF), jnp.int32),
            pltpu.VMEM((2, 2, HALF), jnp.int32),
            pltpu.VMEM((2, ROWS_PER_HALF, HIDDEN), jnp.float32),
            pltpu.VMEM((2, 2, HALF, HIDDEN), jnp.float32),
            pltpu.VMEM((N_NODE + 1, HIDDEN), jnp.float32),
            pltpu.SemaphoreType.DMA,
            pltpu.SemaphoreType.DMA,
            pltpu.SemaphoreType.DMA,
            pltpu.SemaphoreType.DMA,
            pltpu.SemaphoreType.DMA,
        ],
        compiler_params=pltpu.CompilerParams(use_tc_tiling_on_sc=False),
    )
    def k(x_p, in_deg, out_deg, atom_t, in_t, out_t, token,
          out, idxa, idxd, rows, drows, obuf, sem0, sem1, semw, semi0, semi1):
        wid = lax.axis_index("s") * 2 + lax.axis_index("c")
        sems = (sem0, sem1)
        semis = (semi0, semi1)

        def load_idx(g, s, h):
            semi = semis[s]
            for j in range(N_FEAT):
                pltpu.async_copy(x_p.at[j, g, pl.ds(h * HALF, HALF)],
                                 idxa.at[s, j], semi)
            pltpu.async_copy(in_deg.at[g, pl.ds(h * HALF, HALF)],
                             idxd.at[s, 0], semi)
            pltpu.async_copy(out_deg.at[g, pl.ds(h * HALF, HALF)],
                             idxd.at[s, 1], semi)
            for j in range(N_FEAT):
                pltpu.make_async_copy(x_p.at[j, g, pl.ds(h * HALF, HALF)],
                                      idxa.at[s, j], semi).wait()
            pltpu.make_async_copy(in_deg.at[g, pl.ds(h * HALF, HALF)],
                                  idxd.at[s, 0], semi).wait()
            pltpu.make_async_copy(out_deg.at[g, pl.ds(h * HALF, HALF)],
                                  idxd.at[s, 1], semi).wait()

        def fire(s):
            sem = sems[s]
            for j in range(N_FEAT):
                pltpu.async_copy(atom_t.at[idxa.at[s, j]],
                                 rows.at[s, pl.ds(j * HALF, HALF)], sem)
            pltpu.async_copy(in_t.at[idxd.at[s, 0]], drows.at[s, 0], sem)
            pltpu.async_copy(out_t.at[idxd.at[s, 1]], drows.at[s, 1], sem)

        def drain(s):
            sem = sems[s]
            for j in range(N_FEAT):
                pltpu.make_async_copy(atom_t.at[idxa.at[s, j]],
                                      rows.at[s, pl.ds(j * HALF, HALF)], sem).wait()
            pltpu.make_async_copy(in_t.at[idxd.at[s, 0]], drows.at[s, 0], sem).wait()
            pltpu.make_async_copy(out_t.at[idxd.at[s, 1]], drows.at[s, 1], sem).wait()

        def sum_half(s, h):
            base = h * HALF + 1

            def per_node(n, nc):
                cols = []
                for v in range(VPR):
                    sl = pl.ds(v * LANES, LANES)
                    vals = [rows[s, j * HALF + n, sl] for j in range(N_FEAT)]
                    vals.append(drows[s, 0, n, sl])
                    vals.append(drows[s, 1, n, sl])
                    cols.append(vals)
                outs = []
                for vals in cols:
                    while len(vals) > 1:
                        nxt = [vals[i] + vals[i + 1]
                               for i in range(0, len(vals) - 1, 2)]
                        if len(vals) % 2:
                            nxt.append(vals[-1])
                        vals = nxt
                    outs.append(vals[0])
                for v in range(VPR):
                    obuf[base + n, pl.ds(v * LANES, LANES)] = outs[v]
                return nc

            lax.fori_loop(0, HALF, per_node, 0, unroll=2)

        pltpu.sync_copy(token, obuf.at[pl.ds(0, 1)])
        g0 = wid * GRAPHS_PER_WORKER
        load_idx(g0, 0, 0)
        fire(0)

        def per_graph(t, carry):
            g = g0 + t
            load_idx(g, 1, 1)
            fire(1)
            drain(0)

            @pl.when(t > 0)
            def _():
                pltpu.make_async_copy(obuf, out.at[g - 1], semw).wait()

            sum_half(0, 0)

            @pl.when(t < GRAPHS_PER_WORKER - 1)
            def _():
                load_idx(g + 1, 0, 0)
                fire(0)

            drain(1)
            sum_half(1, 1)
            pltpu.async_copy(obuf, out.at[g], semw)
            return carry

        lax.fori_loop(0, GRAPHS_PER_WORKER, per_graph, 0)
        pltpu.make_async_copy(
            obuf, out.at[g0 + GRAPHS_PER_WORKER - 1], semw).wait()

    return k


def kernel(x, in_degree, out_degree, atom_table, in_table, out_table, graph_token_w):
    x_p = jnp.transpose(x.astype(jnp.int32), (2, 0, 1))
    return _sc_kernel()(
        x_p,
        in_degree.astype(jnp.int32),
        out_degree.astype(jnp.int32),
        atom_table,
        in_table,
        out_table,
        graph_token_w,
    )
